# baseline (speedup 1.0000x reference)
import functools

import jax
import jax.numpy as jnp
from jax import lax
from jax.experimental import pallas as pl
from jax.experimental.pallas import tpu as pltpu
from jax.experimental.pallas import tpu_sc as plsc

_NF = 2048
_OUT = 64
_ROWS_PER_TILE = 64
_HW = 128


def _sc_mask_body(hands_hbm, mask_hbm, buf, mbuf):
    wid = lax.axis_index("s") * 2 + lax.axis_index("c")
    blk = wid // 2
    half = wid % 2
    pltpu.sync_copy(hands_hbm.at[:, pl.ds(blk * 128, 128)], buf)
    lane = lax.iota(jnp.int32, 16)
    for cc in range(4):
        c0 = half * 64 + cc * 16

        def fstep(f, acc):
            return jnp.maximum(
                acc, plsc.load_gather(buf, [jnp.full((16,), f, jnp.int32), lane + c0])
            )

        acc = lax.fori_loop(0, _HW, fstep, jnp.zeros((16,), jnp.float32))
        mbuf[pl.ds(cc * 16, 16)] = (acc > 0.0).astype(jnp.int32)
    pltpu.sync_copy(mbuf, mask_hbm.at[pl.ds(wid * _ROWS_PER_TILE, _ROWS_PER_TILE)])


@functools.cache
def _sc_mask():
    return pl.kernel(
        _sc_mask_body,
        out_type=jax.ShapeDtypeStruct((_NF,), jnp.int32),
        mesh=plsc.VectorSubcoreMesh(core_axis_name="c", subcore_axis_name="s"),
        scratch_types=[
            pltpu.VMEM((_HW, 128), jnp.float32),
            pltpu.VMEM((_ROWS_PER_TILE,), jnp.int32),
        ],
        compiler_params=pltpu.CompilerParams(needs_layout_passes=False),
    )


def _tc_pool_body(dl_ref, mask_ref, out_ref, out_i_ref):
    mf = (mask_ref[...] > 0).astype(jnp.float32)
    p = mf
    d = 1
    while d < _NF:
        shifted = jnp.concatenate(
            [jnp.zeros((1, d), jnp.float32), p[:, : _NF - d]], axis=1
        )
        p = p + shifted
        d *= 2
    K = jnp.sum(mf)
    k = p - 1.0
    L = jnp.where(k <= 0.0, 0.0, 2.0 * k + 32.0)
    U = jnp.where(k >= 2047.0, 4159.0, 2.0 * k + 33.0)
    ovec = lax.broadcasted_iota(jnp.int32, (_OUT, 1), 0).astype(jnp.float32) * 65.0
    lo = jnp.maximum(ovec, L)
    hi = jnp.minimum(ovec + 64.0, U)
    c = jnp.maximum(hi - lo + 1.0, 0.0)
    wgm = c * mf * (1.0 / 65.0)
    lk = jnp.where(K == 0.0, 0.0, jnp.where(K >= 2048.0, 4160.0, 2.0 * K + 32.0))
    t = jnp.maximum((ovec + 64.0) - jnp.maximum(ovec, lk) + 1.0, 0.0) * (1.0 / 65.0)
    dl = dl_ref[...]
    out_ref[...] = (
        jnp.dot(wgm, dl, preferred_element_type=jnp.float32) + t * dl[0:1, :]
    )
    jv = lax.broadcasted_iota(jnp.int32, (1, _NF), 1).astype(jnp.float32)
    out_i_ref[...] = jnp.sum(wgm * jv, axis=1)


_tc_pool = pl.pallas_call(
    _tc_pool_body,
    out_shape=[
        jax.ShapeDtypeStruct((_OUT, 357), jnp.float32),
        jax.ShapeDtypeStruct((_OUT,), jnp.float32),
    ],
)


def kernel(data0):
    dl = jnp.concatenate(
        [data0[:, 0:40, :].reshape(_NF, 120), data0[:, 464:543, :].reshape(_NF, 237)],
        axis=1,
    )
    v = jnp.transpose(data0, (2, 1, 0))
    hands = jnp.concatenate(
        [v[:, 468:489, :], v[:, 522:543, :]], axis=1
    ).reshape(126, _NF)
    hands = jnp.concatenate([hands, jnp.zeros((2, _NF), jnp.float32)], axis=0)
    mask = _sc_mask()(hands)
    out, out_i = _tc_pool(dl, mask.reshape(1, _NF))
    data_out = jnp.concatenate([out[:, 0:120], out[:, 132:357]], axis=1)
    return (data_out.reshape(_OUT, 115, 3), out_i)


# speedup vs baseline: 1.2310x; 1.2310x over previous
import functools

import jax
import jax.numpy as jnp
from jax import lax
from jax.experimental import pallas as pl
from jax.experimental.pallas import tpu as pltpu
from jax.experimental.pallas import tpu_sc as plsc

_NF = 2048
_OUT = 64
_ROWS_PER_TILE = 64
_HW = 128


def _sc_mask_body(hands_hbm, mask_hbm, buf, mbuf):
    wid = lax.axis_index("s") * 2 + lax.axis_index("c")
    blk = wid // 2
    half = wid % 2
    pltpu.sync_copy(hands_hbm.at[:, pl.ds(blk * 128, 128)], buf)
    lane = lax.iota(jnp.int32, 16)
    for cc in range(4):
        c0 = half * 64 + cc * 16

        def fstep(f8, acc):
            for k in range(8):
                acc = jnp.maximum(
                    acc,
                    plsc.load_gather(
                        buf, [jnp.full((16,), f8 * 8 + k, jnp.int32), lane + c0]
                    ),
                )
            return acc

        acc = lax.fori_loop(0, _HW // 8, fstep, jnp.zeros((16,), jnp.float32))
        mbuf[pl.ds(cc * 16, 16)] = (acc > 0.0).astype(jnp.int32)
    pltpu.sync_copy(mbuf, mask_hbm.at[pl.ds(wid * _ROWS_PER_TILE, _ROWS_PER_TILE)])


@functools.cache
def _sc_mask():
    return pl.kernel(
        _sc_mask_body,
        out_type=jax.ShapeDtypeStruct((_NF,), jnp.int32),
        mesh=plsc.VectorSubcoreMesh(core_axis_name="c", subcore_axis_name="s"),
        scratch_types=[
            pltpu.VMEM((_HW, 128), jnp.float32),
            pltpu.VMEM((_ROWS_PER_TILE,), jnp.int32),
        ],
        compiler_params=pltpu.CompilerParams(needs_layout_passes=False),
    )


def _tc_pool_body(v3_ref, mask_ref, out_ref, out_i_ref, lips_s, rest_s, sem1, sem2):
    cp1 = pltpu.make_async_copy(v3_ref.at[:, pl.ds(0, 40), :], lips_s, sem1)
    cp1.start()
    cp2 = pltpu.make_async_copy(v3_ref.at[:, pl.ds(464, 79), :], rest_s, sem2)
    cp2.start()
    mf = (mask_ref[...] > 0).astype(jnp.float32)
    p = mf
    d = 1
    while d < _NF:
        shifted = jnp.concatenate(
            [jnp.zeros((1, d), jnp.float32), p[:, : _NF - d]], axis=1
        )
        p = p + shifted
        d *= 2
    K = jnp.sum(mf)
    k = p - 1.0
    L = jnp.where(k <= 0.0, 0.0, 2.0 * k + 32.0)
    U = jnp.where(k >= 2047.0, 4159.0, 2.0 * k + 33.0)
    ovec = lax.broadcasted_iota(jnp.int32, (_OUT, 1), 0).astype(jnp.float32) * 65.0
    lo = jnp.maximum(ovec, L)
    hi = jnp.minimum(ovec + 64.0, U)
    c = jnp.maximum(hi - lo + 1.0, 0.0)
    wgm = c * mf * (1.0 / 65.0)
    lk = jnp.where(K == 0.0, 0.0, jnp.where(K >= 2048.0, 4160.0, 2.0 * K + 32.0))
    ol = lax.broadcasted_iota(jnp.int32, (1, _OUT), 1).astype(jnp.float32) * 65.0
    tt = jnp.maximum((ol + 64.0) - jnp.maximum(ol, lk) + 1.0, 0.0) * (1.0 / 65.0)
    jv = lax.broadcasted_iota(jnp.int32, (1, _NF), 1).astype(jnp.float32)
    out_i_ref[...] = jnp.sum(wgm * jv, axis=1)
    cp1.wait()
    cp2.wait()
    dn = (((1,), (1,)), ((), ()))
    for d in range(3):
        a = lips_s[d]
        b = rest_s[d]
        oa = lax.dot_general(a, wgm, dn, preferred_element_type=jnp.float32)
        ob = lax.dot_general(b, wgm, dn, preferred_element_type=jnp.float32)
        out_ref[d, 0:40, :] = oa + a[:, 0:1] * tt
        out_ref[d, 40:119, :] = ob + b[:, 0:1] * tt


_tc_pool = pl.pallas_call(
    _tc_pool_body,
    in_specs=[
        pl.BlockSpec(memory_space=pltpu.MemorySpace.HBM),
        pl.BlockSpec(memory_space=pltpu.MemorySpace.VMEM),
    ],
    out_shape=[
        jax.ShapeDtypeStruct((3, 119, _OUT), jnp.float32),
        jax.ShapeDtypeStruct((_OUT,), jnp.float32),
    ],
    scratch_shapes=[
        pltpu.VMEM((3, 40, _NF), jnp.float32),
        pltpu.VMEM((3, 79, _NF), jnp.float32),
        pltpu.SemaphoreType.DMA,
        pltpu.SemaphoreType.DMA,
    ],
)


def kernel(data0):
    v3 = jnp.transpose(data0, (2, 1, 0))
    hands = jnp.concatenate(
        [v3[:, 468:489, :], v3[:, 522:543, :]], axis=1
    ).reshape(126, _NF)
    hands = jnp.concatenate([hands, jnp.zeros((2, _NF), jnp.float32)], axis=0)
    mask = _sc_mask()(hands)
    out3, out_i = _tc_pool(v3, mask.reshape(1, _NF))
    o = jnp.transpose(out3, (2, 1, 0))
    data_out = jnp.concatenate([o[:, 0:40, :], o[:, 44:119, :]], axis=1)
    return (data_out, out_i)


# speedup vs baseline: 1.2596x; 1.0232x over previous
import functools

import jax
import jax.numpy as jnp
from jax import lax
from jax.experimental import pallas as pl
from jax.experimental.pallas import tpu as pltpu
from jax.experimental.pallas import tpu_sc as plsc

_NF = 2048
_OUT = 64
_ROWS_PER_TILE = 64
_HW = 128


def _sc_mask_body(hands_hbm, mask_hbm, buf, mbuf):
    wid = lax.axis_index("s") * 2 + lax.axis_index("c")
    blk = wid // 2
    half = wid % 2
    pltpu.sync_copy(hands_hbm.at[:, pl.ds(blk * 128, 128)], buf)
    lane = lax.iota(jnp.int32, 16)
    for cc in range(4):
        c0 = half * 64 + cc * 16

        def fstep(f8, acc):
            for k in range(8):
                acc = jnp.maximum(
                    acc,
                    plsc.load_gather(
                        buf, [jnp.full((16,), f8 * 8 + k, jnp.int32), lane + c0]
                    ),
                )
            return acc

        acc = lax.fori_loop(0, _HW // 8, fstep, jnp.zeros((16,), jnp.float32))
        mbuf[pl.ds(cc * 16, 16)] = (acc > 0.0).astype(jnp.int32)
    pltpu.sync_copy(mbuf, mask_hbm.at[pl.ds(wid * _ROWS_PER_TILE, _ROWS_PER_TILE)])


@functools.cache
def _sc_mask():
    return pl.kernel(
        _sc_mask_body,
        out_type=jax.ShapeDtypeStruct((_NF,), jnp.int32),
        mesh=plsc.VectorSubcoreMesh(core_axis_name="c", subcore_axis_name="s"),
        scratch_types=[
            pltpu.VMEM((_HW, 128), jnp.float32),
            pltpu.VMEM((_ROWS_PER_TILE,), jnp.int32),
        ],
        compiler_params=pltpu.CompilerParams(needs_layout_passes=False),
    )


def _tc_pool_body(v3_ref, mask_ref, out_ref, out_i_ref, lips_s, rest_s, sem1, sem2):
    cp1 = pltpu.make_async_copy(v3_ref.at[:, pl.ds(0, 40), :], lips_s, sem1)
    cp1.start()
    cp2 = pltpu.make_async_copy(v3_ref.at[:, pl.ds(464, 79), :], rest_s, sem2)
    cp2.start()
    mf = (mask_ref[...] > 0).astype(jnp.float32)
    p = mf
    d = 1
    while d < _NF:
        shifted = jnp.concatenate(
            [jnp.zeros((1, d), jnp.float32), p[:, : _NF - d]], axis=1
        )
        p = p + shifted
        d *= 2
    K = jnp.sum(mf)
    k = p - 1.0
    L = jnp.where(k <= 0.0, 0.0, 2.0 * k + 32.0)
    U = jnp.where(k >= 2047.0, 4159.0, 2.0 * k + 33.0)
    ovec = lax.broadcasted_iota(jnp.int32, (_OUT, 1), 0).astype(jnp.float32) * 65.0
    lo = jnp.maximum(ovec, L)
    hi = jnp.minimum(ovec + 64.0, U)
    c = jnp.maximum(hi - lo + 1.0, 0.0)
    wgm = c * mf * (1.0 / 65.0)
    lk = jnp.where(K == 0.0, 0.0, jnp.where(K >= 2048.0, 4160.0, 2.0 * K + 32.0))
    ol = lax.broadcasted_iota(jnp.int32, (1, _OUT), 1).astype(jnp.float32) * 65.0
    tt = jnp.maximum((ol + 64.0) - jnp.maximum(ol, lk) + 1.0, 0.0) * (1.0 / 65.0)
    jv = lax.broadcasted_iota(jnp.int32, (1, _NF), 1).astype(jnp.float32)
    out_i_ref[...] = jnp.sum(wgm * jv, axis=1)
    cp1.wait()
    cp2.wait()
    dn = (((1,), (1,)), ((), ()))
    for d in range(3):
        a = lips_s[d]
        b = rest_s[d]
        oa = lax.dot_general(a, wgm, dn, preferred_element_type=jnp.float32)
        ob = lax.dot_general(b, wgm, dn, preferred_element_type=jnp.float32)
        out_ref[d, 0:40, :] = oa + a[:, 0:1] * tt
        out_ref[d, 40:115, :] = (ob + b[:, 0:1] * tt)[4:79, :]


_tc_pool = pl.pallas_call(
    _tc_pool_body,
    in_specs=[
        pl.BlockSpec(memory_space=pltpu.MemorySpace.HBM),
        pl.BlockSpec(memory_space=pltpu.MemorySpace.VMEM),
    ],
    out_shape=[
        jax.ShapeDtypeStruct((3, 115, _OUT), jnp.float32),
        jax.ShapeDtypeStruct((_OUT,), jnp.float32),
    ],
    scratch_shapes=[
        pltpu.VMEM((3, 40, _NF), jnp.float32),
        pltpu.VMEM((3, 79, _NF), jnp.float32),
        pltpu.SemaphoreType.DMA,
        pltpu.SemaphoreType.DMA,
    ],
)


def kernel(data0):
    v3 = jnp.transpose(data0, (2, 1, 0))
    hands = jnp.concatenate(
        [v3[:, 468:489, :], v3[:, 522:543, :]], axis=1
    ).reshape(126, _NF)
    hands = jnp.concatenate([hands, jnp.zeros((2, _NF), jnp.float32)], axis=0)
    mask = _sc_mask()(hands)
    out3, out_i = _tc_pool(v3, mask.reshape(1, _NF))
    return (jnp.transpose(out3, (2, 1, 0)), out_i)


# speedup vs baseline: 1.2747x; 1.0120x over previous
import functools

import jax
import jax.numpy as jnp
from jax import lax
from jax.experimental import pallas as pl
from jax.experimental.pallas import tpu as pltpu
from jax.experimental.pallas import tpu_sc as plsc

_NF = 2048
_OUT = 64
_ROWS_PER_TILE = 64
_HW = 128


def _sc_mask_body(hands_hbm, mask_hbm, buf, mbuf):
    wid = lax.axis_index("s") * 2 + lax.axis_index("c")
    blk = wid // 2
    half = wid % 2
    pltpu.sync_copy(hands_hbm.at[:, pl.ds(blk * 128, 128)], buf)
    lane = lax.iota(jnp.int32, 16)
    for cc in range(4):
        c0 = half * 64 + cc * 16

        def fstep(f7, acc):
            for k in range(7):
                acc = jnp.maximum(
                    acc,
                    plsc.load_gather(
                        buf, [jnp.full((16,), f7 * 7 + k, jnp.int32), lane + c0]
                    ),
                )
            return acc

        acc = lax.fori_loop(0, 18, fstep, jnp.zeros((16,), jnp.float32))
        mbuf[pl.ds(cc * 16, 16)] = (acc > 0.0).astype(jnp.int32)
    pltpu.sync_copy(mbuf, mask_hbm.at[pl.ds(wid * _ROWS_PER_TILE, _ROWS_PER_TILE)])


@functools.cache
def _sc_mask():
    return pl.kernel(
        _sc_mask_body,
        out_type=jax.ShapeDtypeStruct((_NF,), jnp.int32),
        mesh=plsc.VectorSubcoreMesh(core_axis_name="c", subcore_axis_name="s"),
        scratch_types=[
            pltpu.VMEM((126, 128), jnp.float32),
            pltpu.VMEM((_ROWS_PER_TILE,), jnp.int32),
        ],
        compiler_params=pltpu.CompilerParams(needs_layout_passes=False),
    )


def _tc_pool_body(v3_ref, mask_ref, out_ref, out_i_ref, lips_s, rest_s, sem1, sem2):
    cp1 = pltpu.make_async_copy(v3_ref.at[:, pl.ds(0, 40), :], lips_s, sem1)
    cp1.start()
    cp2 = pltpu.make_async_copy(v3_ref.at[:, pl.ds(464, 79), :], rest_s, sem2)
    cp2.start()
    mf = (mask_ref[...] > 0).astype(jnp.float32)
    p = mf
    d = 1
    while d < _NF:
        shifted = jnp.concatenate(
            [jnp.zeros((1, d), jnp.float32), p[:, : _NF - d]], axis=1
        )
        p = p + shifted
        d *= 2
    K = jnp.sum(mf)
    k = p - 1.0
    L = jnp.where(k <= 0.0, 0.0, 2.0 * k + 32.0)
    U = jnp.where(k >= 2047.0, 4159.0, 2.0 * k + 33.0)
    ovec = lax.broadcasted_iota(jnp.int32, (_OUT, 1), 0).astype(jnp.float32) * 65.0
    lo = jnp.maximum(ovec, L)
    hi = jnp.minimum(ovec + 64.0, U)
    c = jnp.maximum(hi - lo + 1.0, 0.0)
    wgm = c * mf * (1.0 / 65.0)
    lk = jnp.where(K == 0.0, 0.0, jnp.where(K >= 2048.0, 4160.0, 2.0 * K + 32.0))
    ol = lax.broadcasted_iota(jnp.int32, (1, _OUT), 1).astype(jnp.float32) * 65.0
    tt = jnp.maximum((ol + 64.0) - jnp.maximum(ol, lk) + 1.0, 0.0) * (1.0 / 65.0)
    jv = lax.broadcasted_iota(jnp.int32, (1, _NF), 1).astype(jnp.float32)
    out_i_ref[...] = jnp.sum(wgm * jv, axis=1)
    cp1.wait()
    cp2.wait()
    dn = (((1,), (1,)), ((), ()))
    for d in range(3):
        a = lips_s[d]
        b = rest_s[d]
        oa = lax.dot_general(a, wgm, dn, preferred_element_type=jnp.float32)
        ob = lax.dot_general(b, wgm, dn, preferred_element_type=jnp.float32)
        out_ref[d, 0:40, :] = oa + a[:, 0:1] * tt
        out_ref[d, 40:115, :] = (ob + b[:, 0:1] * tt)[4:79, :]


_tc_pool = pl.pallas_call(
    _tc_pool_body,
    in_specs=[
        pl.BlockSpec(memory_space=pltpu.MemorySpace.HBM),
        pl.BlockSpec(memory_space=pltpu.MemorySpace.VMEM),
    ],
    out_shape=[
        jax.ShapeDtypeStruct((3, 115, _OUT), jnp.float32),
        jax.ShapeDtypeStruct((_OUT,), jnp.float32),
    ],
    scratch_shapes=[
        pltpu.VMEM((3, 40, _NF), jnp.float32),
        pltpu.VMEM((3, 79, _NF), jnp.float32),
        pltpu.SemaphoreType.DMA,
        pltpu.SemaphoreType.DMA,
    ],
)


def kernel(data0):
    v3 = jnp.transpose(data0, (2, 1, 0))
    hands = jnp.concatenate(
        [v3[:, 468:489, :], v3[:, 522:543, :]], axis=1
    ).reshape(126, _NF)
    mask = _sc_mask()(hands)
    out3, out_i = _tc_pool(v3, mask.reshape(1, _NF))
    return (jnp.transpose(out3, (2, 1, 0)), out_i)
